# baseline (speedup 1.0000x reference)
import functools

import jax
import jax.numpy as jnp
from jax import lax
from jax.experimental import pallas as pl
from jax.experimental.pallas import tpu as pltpu
from jax.experimental.pallas import tpu_sc as plsc

B = 4096
N_AGENTS = 8
INP = 256
HID = 512
ROLE = 64
NROLES = 512
STATE = 256
M = B * N_AGENTS

ROWS = 4096


def _fused_body(x_ref, w0_ref, b0_ref, w1_ref, b1_ref, w2_ref, b2_ref,
                c_ref, w3_ref, b3_ref, w4_ref, b4_ref, w5_ref, b5_ref,
                rec_ref, ze_ref, idx_ref):
    x = x_ref[...]
    h = jnp.maximum(jnp.dot(x, w0_ref[...]) + b0_ref[...], 0.0)
    h = jnp.maximum(jnp.dot(h, w1_ref[...]) + b1_ref[...], 0.0)
    ze = jnp.dot(h, w2_ref[...]) + b2_ref[...]
    ze_ref[...] = ze

    c = c_ref[...]
    d = (jnp.sum(ze * ze, axis=1, keepdims=True)
         - 2.0 * jnp.dot(ze, c.T)
         + jnp.sum(c * c, axis=1)[None, :])
    idx = jnp.argmin(d, axis=1).astype(jnp.int32)
    idx_ref[...] = idx.reshape(1, ROWS)
    onehot = (idx[:, None] == lax.broadcasted_iota(jnp.int32, (ROWS, NROLES), 1)
              ).astype(jnp.float32)
    zq = jnp.dot(onehot, c)

    r1 = jnp.maximum(
        jnp.dot(zq.astype(jnp.bfloat16), w3_ref[...].astype(jnp.bfloat16),
                preferred_element_type=jnp.float32) + b3_ref[...], 0.0)
    r1w = r1.reshape(ROWS // N_AGENTS, N_AGENTS * HID)
    h4 = jnp.maximum(jnp.dot(r1w, w4_ref[...]) + b4_ref[...], 0.0)
    rec_ref[...] = jnp.dot(h4, w5_ref[...]) + b5_ref[...]


def _full(shape):
    return pl.BlockSpec(shape, lambda i: (0,) * len(shape))


_SC_NC = 2
_SC_NS = 16
_SC_CHUNK = M // (_SC_NC * _SC_NS)


@functools.partial(
    pl.kernel,
    mesh=plsc.VectorSubcoreMesh(core_axis_name="c", subcore_axis_name="s"),
    out_type=jax.ShapeDtypeStruct((M, 2 * ROLE), jnp.float32),
    scratch_types=[
        pltpu.VMEM((_SC_CHUNK // 2,), jnp.int32),
        pltpu.VMEM((_SC_CHUNK // 2, 2 * ROLE), jnp.float32),
        pltpu.SemaphoreType.DMA,
    ],
)
def _sc_role_gather(table_hbm, idx_hbm, out_hbm, idx_v, rows_v, sem):
    wid = lax.axis_index("s") * _SC_NC + lax.axis_index("c")
    half = _SC_CHUNK // 2
    for r in range(2):
        base = wid * _SC_CHUNK + r * half
        pltpu.sync_copy(idx_hbm.at[pl.ds(base, half)], idx_v)
        pltpu.async_copy(table_hbm.at[idx_v], rows_v, sem).wait()
        pltpu.sync_copy(rows_v, out_hbm.at[pl.ds(base, half)])


def kernel(inputs, W0, b0, W1, b1, W2, b2, codebook, W3, b3, W4, b4, W5, b5):
    grid = (M // ROWS,)
    rec, ze, idx = pl.pallas_call(
        _fused_body,
        grid=grid,
        in_specs=[
            pl.BlockSpec((ROWS, INP), lambda i: (i, 0)),
            _full((INP, HID)), _full((1, HID)),
            _full((HID, HID)), _full((1, HID)),
            _full((HID, ROLE)), _full((1, ROLE)),
            _full((NROLES, ROLE)),
            _full((ROLE, HID)), _full((1, HID)),
            _full((N_AGENTS * HID, HID)), _full((1, HID)),
            _full((HID, STATE)), _full((1, STATE)),
        ],
        out_specs=[
            pl.BlockSpec((ROWS // N_AGENTS, STATE), lambda i: (i, 0)),
            pl.BlockSpec((ROWS, ROLE), lambda i: (i, 0)),
            pl.BlockSpec((1, ROWS), lambda i: (0, i)),
        ],
        out_shape=[
            jax.ShapeDtypeStruct((M // N_AGENTS, STATE), jnp.float32),
            jax.ShapeDtypeStruct((M, ROLE), jnp.float32),
            jax.ShapeDtypeStruct((1, M), jnp.int32),
        ],
    )(
        inputs,
        W0, b0.reshape(1, HID),
        W1, b1.reshape(1, HID),
        W2, b2.reshape(1, ROLE),
        codebook,
        W3, b3.reshape(1, HID),
        W4, b4.reshape(1, HID),
        W5, b5.reshape(1, STATE),
    )
    table = jnp.pad(codebook, ((0, 0), (0, ROLE)))
    role_emb = _sc_role_gather(table, idx.reshape(M))[:, :ROLE]
    return rec, ze, role_emb


# speedup vs baseline: 8.6540x; 8.6540x over previous
import functools

import jax
import jax.numpy as jnp
from jax import lax
from jax.experimental import pallas as pl
from jax.experimental.pallas import tpu as pltpu

B = 4096
N_AGENTS = 8
INP = 256
HID = 512
ROLE = 64
NROLES = 512
STATE = 256
M = B * N_AGENTS

ROWS = 4096


def _fused_body(x_ref, w0_ref, b0_ref, w1_ref, b1_ref, w2_ref, b2_ref,
                c_ref, w3_ref, b3_ref, w4_ref, b4_ref, w5_ref, b5_ref,
                rec_ref, ze_ref, zq_ref):
    x = x_ref[...]
    h = jnp.maximum(jnp.dot(x, w0_ref[...]) + b0_ref[...], 0.0)
    h = jnp.maximum(jnp.dot(h, w1_ref[...]) + b1_ref[...], 0.0)
    ze = jnp.dot(h, w2_ref[...]) + b2_ref[...]
    ze_ref[...] = ze

    c = c_ref[...]
    d = (jnp.sum(ze * ze, axis=1, keepdims=True)
         - 2.0 * jnp.dot(ze, c.T)
         + jnp.sum(c * c, axis=1)[None, :])
    idx = jnp.argmin(d, axis=1).astype(jnp.int32)
    onehot = (idx[:, None] == lax.broadcasted_iota(jnp.int32, (ROWS, NROLES), 1)
              ).astype(jnp.float32)
    zq = jnp.dot(onehot, c)
    zq_ref[...] = zq

    r1 = jnp.maximum(
        jnp.dot(zq.astype(jnp.bfloat16), w3_ref[...].astype(jnp.bfloat16),
                preferred_element_type=jnp.float32) + b3_ref[...], 0.0)
    r1w = r1.reshape(ROWS // N_AGENTS, N_AGENTS * HID)
    h4 = jnp.maximum(jnp.dot(r1w, w4_ref[...]) + b4_ref[...], 0.0)
    rec_ref[...] = jnp.dot(h4, w5_ref[...]) + b5_ref[...]


def _full(shape):
    return pl.BlockSpec(shape, lambda i: (0,) * len(shape))


def kernel(inputs, W0, b0, W1, b1, W2, b2, codebook, W3, b3, W4, b4, W5, b5):
    grid = (M // ROWS,)
    rec, ze, zq = pl.pallas_call(
        _fused_body,
        grid=grid,
        in_specs=[
            pl.BlockSpec((ROWS, INP), lambda i: (i, 0)),
            _full((INP, HID)), _full((1, HID)),
            _full((HID, HID)), _full((1, HID)),
            _full((HID, ROLE)), _full((1, ROLE)),
            _full((NROLES, ROLE)),
            _full((ROLE, HID)), _full((1, HID)),
            _full((N_AGENTS * HID, HID)), _full((1, HID)),
            _full((HID, STATE)), _full((1, STATE)),
        ],
        out_specs=[
            pl.BlockSpec((ROWS // N_AGENTS, STATE), lambda i: (i, 0)),
            pl.BlockSpec((ROWS, ROLE), lambda i: (i, 0)),
            pl.BlockSpec((ROWS, ROLE), lambda i: (i, 0)),
        ],
        out_shape=[
            jax.ShapeDtypeStruct((M // N_AGENTS, STATE), jnp.float32),
            jax.ShapeDtypeStruct((M, ROLE), jnp.float32),
            jax.ShapeDtypeStruct((M, ROLE), jnp.float32),
        ],
    )(
        inputs,
        W0, b0.reshape(1, HID),
        W1, b1.reshape(1, HID),
        W2, b2.reshape(1, ROLE),
        codebook,
        W3, b3.reshape(1, HID),
        W4, b4.reshape(1, HID),
        W5, b5.reshape(1, STATE),
    )
    role_emb = zq
    return rec, ze, role_emb
